# baseline (speedup 1.0000x reference)
import functools

import jax
import jax.numpy as jnp
from jax import lax
from jax.experimental import pallas as pl
from jax.experimental.pallas import tpu as pltpu
from jax.experimental.pallas import tpu_sc as plsc

N = 10000
E = 320000
D = 128
K = 3

NC = 2
NS = 16
NW = NC * NS
EPW = E // NW
CHUNK = 80
NCHUNK = EPW // CHUNK
N_PAD = 10240
WCH = 80
RPT = N_PAD // NS
RCH = RPT // WCH

_mesh = plsc.VectorSubcoreMesh(core_axis_name="c", subcore_axis_name="s")


@functools.partial(
    pl.kernel,
    out_type=jax.ShapeDtypeStruct((NW, N_PAD), jnp.float32),
    mesh=_mesh,
    scratch_types=[
        pltpu.VMEM((EPW // 16, 16), jnp.int32),
        pltpu.VMEM((N_PAD,), jnp.float32),
    ],
    compiler_params=pltpu.CompilerParams(needs_layout_passes=False),
)
def _sc_degree(dst_hbm, out_hbm, dst_v, deg_v):
    cid = lax.axis_index("c")
    sid = lax.axis_index("s")
    wid = cid * NS + sid
    pltpu.sync_copy(dst_hbm.at[wid], dst_v)

    zeros = jnp.zeros((16,), jnp.float32)

    def zbody(i, _):
        deg_v[pl.ds(i * 16, 16)] = zeros
        return 0

    lax.fori_loop(0, N_PAD // 16, zbody, 0)

    ones = jnp.ones((16,), jnp.float32)

    def body(j, _):
        idx = dst_v[j, pl.ds(0, 16)]
        plsc.addupdate_scatter(deg_v, [idx], ones)
        return 0

    lax.fori_loop(0, EPW // 16, body, 0)
    pltpu.sync_copy(deg_v, out_hbm.at[wid])


@functools.partial(
    pl.kernel,
    out_type=[
        jax.ShapeDtypeStruct((NW, N_PAD), jnp.float32),
        jax.ShapeDtypeStruct((NW, N_PAD), jnp.float32),
    ],
    mesh=_mesh,
    scratch_types=[
        pltpu.VMEM((EPW,), jnp.int32),
        pltpu.VMEM((EPW,), jnp.int32),
        pltpu.VMEM((N_PAD,), jnp.float32),
        pltpu.VMEM((N_PAD,), jnp.float32),
        pltpu.VMEM((N_PAD,), jnp.float32),
        pltpu.VMEM((N_PAD,), jnp.float32),
    ],
    compiler_params=pltpu.CompilerParams(needs_layout_passes=False),
)
def _sc_scalar_sum(src_hbm, dst_hbm, p_hbm, q_hbm, outp_hbm, outq_hbm,
                   src_v, dst_v, p_v, q_v, ap_v, aq_v):
    cid = lax.axis_index("c")
    sid = lax.axis_index("s")
    wid = cid * NS + sid
    pltpu.sync_copy(src_hbm.at[wid], src_v)
    pltpu.sync_copy(dst_hbm.at[wid], dst_v)
    pltpu.sync_copy(p_hbm, p_v)
    pltpu.sync_copy(q_hbm, q_v)

    zeros = jnp.zeros((16,), jnp.float32)

    def zbody(i, _):
        ap_v[pl.ds(i * 16, 16)] = zeros
        aq_v[pl.ds(i * 16, 16)] = zeros
        return 0

    lax.fori_loop(0, N_PAD // 16, zbody, 0)

    def body(j, _):
        o = pl.multiple_of(j * 16, 8)
        s_idx = src_v[pl.ds(o, 16)]
        d_idx = dst_v[pl.ds(o, 16)]
        plsc.addupdate_scatter(ap_v, [d_idx], plsc.load_gather(p_v, [s_idx]))
        plsc.addupdate_scatter(aq_v, [d_idx], plsc.load_gather(q_v, [s_idx]))
        return 0

    lax.fori_loop(0, EPW // 16, body, 0)
    pltpu.sync_copy(ap_v, outp_hbm.at[wid])
    pltpu.sync_copy(aq_v, outq_hbm.at[wid])


@functools.partial(
    pl.kernel,
    out_type=jax.ShapeDtypeStruct((NC, N_PAD, D), jnp.float32),
    mesh=_mesh,
    scratch_types=[
        pltpu.VMEM((EPW,), jnp.int32),
        pltpu.VMEM((NCHUNK, CHUNK), jnp.int32),
        pltpu.VMEM((2, CHUNK, D), jnp.float32),
        pltpu.VMEM_SHARED((N_PAD, D), jnp.float32),
        pltpu.SemaphoreType.DMA,
        pltpu.SemaphoreType.DMA,
    ],
)
def _sc_edge_sum(src_hbm, dst_hbm, xn_hbm, out_hbm, src_v, dst_v, rows_v,
                 acc_sh, gsem, ssem):
    cid = lax.axis_index("c")
    sid = lax.axis_index("s")
    wid = cid * NS + sid

    pltpu.sync_copy(src_hbm.at[wid], src_v)
    pltpu.sync_copy(dst_hbm.at[wid], dst_v)

    zeros = jnp.zeros((16,), jnp.float32)

    def zbody(i, _):
        for k in range(D // 16):
            rows_v[0, i, pl.ds(k * 16, 16)] = zeros
        return 0

    lax.fori_loop(0, WCH, zbody, 0)
    zrows = rows_v.at[0, pl.ds(0, WCH)]
    for k in range(RCH):
        pltpu.sync_copy(zrows, acc_sh.at[pl.ds(sid * RPT + k * WCH, WCH)])
    plsc.subcore_barrier()

    def src_sl(j):
        return src_v.at[pl.ds(pl.multiple_of(j * CHUNK, 8), CHUNK)]

    def gather(j, p):
        return pltpu.make_async_copy(xn_hbm.at[src_sl(j)], rows_v.at[p],
                                     gsem)

    def scatter(j, p):
        return pltpu.make_async_copy(rows_v.at[p], acc_sh.at[dst_v.at[j]],
                                     ssem)

    gather(0, 0).start()

    def body(j, _):
        p = lax.rem(j, 2)
        gather(j, p).wait()
        pltpu.async_copy(rows_v.at[p], acc_sh.at[dst_v.at[j]], ssem,
                         add=True)

        @pl.when(j >= 1)
        def _():
            scatter(j - 1, 1 - p).wait()

        @pl.when(j < NCHUNK - 1)
        def _():
            gather(j + 1, 1 - p).start()

        return 0

    lax.fori_loop(0, NCHUNK, body, 0)
    scatter(NCHUNK - 1, (NCHUNK - 1) % 2).wait()
    plsc.subcore_barrier()

    for k in range(RCH):
        base = sid * RPT + k * WCH
        pltpu.sync_copy(acc_sh.at[pl.ds(base, WCH)], zrows)
        pltpu.sync_copy(zrows, out_hbm.at[cid, pl.ds(base, WCH)])


_BLK = 1024
_GRID = N_PAD // _BLK


_COL = lambda i: (i, 0)
_CSPEC = pl.BlockSpec((_BLK, 1), _COL)
_PSPEC = pl.BlockSpec((NW, _BLK), lambda i: (0, i))
_CSHAPE = jax.ShapeDtypeStruct((N_PAD, 1), jnp.float32)


def _tc_norm_body(w_ref, deg_ref, nrm_ref, p1_ref):
    deg = jnp.sum(deg_ref[...], axis=0)[:, None] + 1.0
    norm = lax.rsqrt(deg)
    nrm_ref[...] = norm
    p1_ref[...] = norm * w_ref[...]


def _tc_norm(weight_pad, deg_parts):
    return pl.pallas_call(
        _tc_norm_body,
        grid=(_GRID,),
        in_specs=[_CSPEC, _PSPEC],
        out_specs=[_CSPEC, _CSPEC],
        out_shape=[_CSHAPE, _CSHAPE],
    )(weight_pad, deg_parts)


def _tc_scal_combine_body(sp_ref, sq_ref, p1_ref, q1_ref, nrm_ref,
                          a1_ref, b1_ref, p2_ref, q2_ref):
    norm = nrm_ref[...]
    a1 = -norm * (jnp.sum(sp_ref[...], axis=0)[:, None] + p1_ref[...])
    b1 = -norm * (jnp.sum(sq_ref[...], axis=0)[:, None] + q1_ref[...])
    a1_ref[...] = a1
    b1_ref[...] = b1
    p2_ref[...] = norm * a1
    q2_ref[...] = norm * b1


def _tc_scal_combine(sp_parts, sq_parts, p1, q1, norm):
    return pl.pallas_call(
        _tc_scal_combine_body,
        grid=(_GRID,),
        in_specs=[_PSPEC, _PSPEC, _CSPEC, _CSPEC, _CSPEC],
        out_specs=[_CSPEC, _CSPEC, _CSPEC, _CSPEC],
        out_shape=[_CSHAPE, _CSHAPE, _CSHAPE, _CSHAPE],
    )(sp_parts, sq_parts, p1, q1, norm)


def _tc_conv1_body(sp_ref, sq_ref, p2_ref, q2_ref, w_ref, nrm_ref, a1_ref,
                   b1_ref, win_ref, bin_ref, cw_ref, cb_ref,
                   emb_ref, xnn_ref):
    norm = nrm_ref[...]
    ah2 = norm * (jnp.sum(sp_ref[...], axis=0)[:, None] + p2_ref[...])
    bh2 = norm * (jnp.sum(sq_ref[...], axis=0)[:, None] + q2_ref[...])
    a2 = -2.0 * ah2 - w_ref[...]
    b2 = -2.0 * bh2 - 1.0
    win, bin_ = win_ref[...], bin_ref[...]
    tx0 = w_ref[...] * win + bin_
    tx1 = a1_ref[...] * win + b1_ref[...] * bin_
    tx2 = a2 * win + b2 * bin_
    rst = (jnp.dot(tx0, cw_ref[0], preferred_element_type=jnp.float32)
           + jnp.dot(tx1, cw_ref[1], preferred_element_type=jnp.float32)
           + jnp.dot(tx2, cw_ref[2], preferred_element_type=jnp.float32)
           + cb_ref[...])
    emb = jnp.where(rst >= 0.0, rst, 0.01 * rst)
    emb_ref[...] = emb
    xnn_ref[...] = emb * norm


def _tc_conv1(sp_parts, sq_parts, p2, q2, weight_pad, norm, a1, b1,
              w_in, b_in, cw, cb):
    return pl.pallas_call(
        _tc_conv1_body,
        grid=(_GRID,),
        in_specs=[_PSPEC, _PSPEC, _CSPEC, _CSPEC, _CSPEC, _CSPEC, _CSPEC,
                  _CSPEC,
                  pl.BlockSpec((1, D), lambda i: (0, 0)),
                  pl.BlockSpec((1, D), lambda i: (0, 0)),
                  pl.BlockSpec((K, D, D), lambda i: (0, 0, 0)),
                  pl.BlockSpec((1, D), lambda i: (0, 0))],
        out_specs=[pl.BlockSpec((_BLK, D), _COL),
                   pl.BlockSpec((_BLK, D), _COL)],
        out_shape=[
            jax.ShapeDtypeStruct((N_PAD, D), jnp.float32),
            jax.ShapeDtypeStruct((N_PAD, D), jnp.float32),
        ],
    )(sp_parts, sq_parts, p2, q2, weight_pad, norm, a1, b1, w_in, b_in,
      cw, cb)


def _tc_combine_body(s_ref, xn_ref, nrm_ref, tx1_ref, xn1_ref):
    s = s_ref[0] + s_ref[1] + xn_ref[...]
    norm = nrm_ref[...]
    tx1 = -(s * norm)
    tx1_ref[...] = tx1
    xn1_ref[...] = tx1 * norm


def _tc_combine(s_parts, xn, norm):
    return pl.pallas_call(
        _tc_combine_body,
        grid=(_GRID,),
        in_specs=[
            pl.BlockSpec((NC, _BLK, D), lambda i: (0, i, 0)),
            pl.BlockSpec((_BLK, D), lambda i: (i, 0)),
            pl.BlockSpec((_BLK, 1), lambda i: (i, 0)),
        ],
        out_specs=[
            pl.BlockSpec((_BLK, D), lambda i: (i, 0)),
            pl.BlockSpec((_BLK, D), lambda i: (i, 0)),
        ],
        out_shape=[
            jax.ShapeDtypeStruct((N_PAD, D), jnp.float32),
            jax.ShapeDtypeStruct((N_PAD, D), jnp.float32),
        ],
    )(s_parts, xn, norm)


def _tc_conv_final_body(s_ref, xn1_ref, x_ref, tx1_ref, nrm_ref, w_ref, b_ref,
                        wout_ref, bout_ref, out_ref):
    norm = nrm_ref[...]
    tx2 = -2.0 * ((s_ref[0] + s_ref[1] + xn1_ref[...]) * norm) - x_ref[...]
    rst = (jnp.dot(x_ref[...], w_ref[0], preferred_element_type=jnp.float32)
           + jnp.dot(tx1_ref[...], w_ref[1], preferred_element_type=jnp.float32)
           + jnp.dot(tx2, w_ref[2], preferred_element_type=jnp.float32)
           + b_ref[...])
    out_ref[...] = (jnp.dot(rst, wout_ref[...],
                            preferred_element_type=jnp.float32)
                    + bout_ref[...])


def _tc_conv_final(s_parts, xn1, x, tx1, norm, w, b, w_out, b_out):
    return pl.pallas_call(
        _tc_conv_final_body,
        grid=(_GRID,),
        in_specs=[
            pl.BlockSpec((NC, _BLK, D), lambda i: (0, i, 0)),
            pl.BlockSpec((_BLK, D), lambda i: (i, 0)),
            pl.BlockSpec((_BLK, D), lambda i: (i, 0)),
            pl.BlockSpec((_BLK, D), lambda i: (i, 0)),
            pl.BlockSpec((_BLK, 1), lambda i: (i, 0)),
            pl.BlockSpec((K, D, D), lambda i: (0, 0, 0)),
            pl.BlockSpec((1, D), lambda i: (0, 0)),
            pl.BlockSpec((D, 1), lambda i: (0, 0)),
            pl.BlockSpec((1, 1), lambda i: (0, 0)),
        ],
        out_specs=pl.BlockSpec((_BLK, 1), lambda i: (i, 0)),
        out_shape=jax.ShapeDtypeStruct((N_PAD, 1), jnp.float32),
    )(s_parts, xn1, x, tx1, norm, w, b, w_out, b_out)


def kernel(weight, edge_index, W_in, b_in, cheb_w, cheb_b, W_out, b_out):
    src = edge_index[0].reshape(NW, EPW)
    dst = edge_index[1].reshape(NW, NCHUNK, CHUNK)
    dst_flat = edge_index[1].reshape(NW, EPW)
    dst_deg = edge_index[1].reshape(NW, EPW // 16, 16)
    weight_pad = jnp.pad(weight, (0, N_PAD - N)).reshape(N_PAD, 1)
    w_in = W_in.reshape(1, D)
    b_in = b_in.reshape(1, D)

    deg_parts = _sc_degree(dst_deg)
    norm, p1 = _tc_norm(weight_pad, deg_parts)

    sp1, sq1 = _sc_scalar_sum(src, dst_flat, p1.reshape(N_PAD),
                              norm.reshape(N_PAD))
    a1, b1, p2, q2 = _tc_scal_combine(sp1, sq1, p1, norm, norm)
    sp2, sq2 = _sc_scalar_sum(src, dst_flat, p2.reshape(N_PAD),
                              q2.reshape(N_PAD))
    emb, xn0b = _tc_conv1(sp2, sq2, p2, q2, weight_pad, norm, a1, b1,
                          w_in, b_in, cheb_w[0], cheb_b[0].reshape(1, D))

    s3 = _sc_edge_sum(src, dst, xn0b)
    tx1b, xn1b = _tc_combine(s3, xn0b, norm)
    s4 = _sc_edge_sum(src, dst, xn1b)
    logits = _tc_conv_final(s4, xn1b, emb, tx1b, norm, cheb_w[1],
                            cheb_b[1].reshape(1, D), W_out,
                            b_out.reshape(1, 1))
    return logits[:N]


# speedup vs baseline: 2.0743x; 2.0743x over previous
import functools

import jax
import jax.numpy as jnp
from jax import lax
from jax.experimental import pallas as pl
from jax.experimental.pallas import tpu as pltpu
from jax.experimental.pallas import tpu_sc as plsc

N = 10000
E = 320000
D = 128
K = 3

NC = 2
NS = 16
NW = NC * NS
EPW = E // NW
N_PAD = 10240

_mesh = plsc.VectorSubcoreMesh(core_axis_name="c", subcore_axis_name="s")


@functools.partial(
    pl.kernel,
    out_type=jax.ShapeDtypeStruct((NW, N_PAD), jnp.float32),
    mesh=_mesh,
    scratch_types=[
        pltpu.VMEM((EPW // 16, 16), jnp.int32),
        pltpu.VMEM((N_PAD,), jnp.float32),
    ],
    compiler_params=pltpu.CompilerParams(needs_layout_passes=False),
)
def _sc_degree(dst_hbm, out_hbm, dst_v, deg_v):
    cid = lax.axis_index("c")
    sid = lax.axis_index("s")
    wid = cid * NS + sid
    pltpu.sync_copy(dst_hbm.at[wid], dst_v)

    zeros = jnp.zeros((16,), jnp.float32)

    def zbody(i, _):
        deg_v[pl.ds(i * 16, 16)] = zeros
        return 0

    lax.fori_loop(0, N_PAD // 16, zbody, 0)

    ones = jnp.ones((16,), jnp.float32)

    def body(j, _):
        idx = dst_v[j, pl.ds(0, 16)]
        plsc.addupdate_scatter(deg_v, [idx], ones)
        return 0

    lax.fori_loop(0, EPW // 16, body, 0)
    pltpu.sync_copy(deg_v, out_hbm.at[wid])


@functools.partial(
    pl.kernel,
    out_type=[
        jax.ShapeDtypeStruct((NW, N_PAD), jnp.float32),
        jax.ShapeDtypeStruct((NW, N_PAD), jnp.float32),
    ],
    mesh=_mesh,
    scratch_types=[
        pltpu.VMEM((EPW,), jnp.int32),
        pltpu.VMEM((EPW,), jnp.int32),
        pltpu.VMEM((N_PAD,), jnp.float32),
        pltpu.VMEM((N_PAD,), jnp.float32),
        pltpu.VMEM((N_PAD,), jnp.float32),
        pltpu.VMEM((N_PAD,), jnp.float32),
    ],
    compiler_params=pltpu.CompilerParams(needs_layout_passes=False),
)
def _sc_scalar_sum(src_hbm, dst_hbm, p_hbm, q_hbm, outp_hbm, outq_hbm,
                   src_v, dst_v, p_v, q_v, ap_v, aq_v):
    cid = lax.axis_index("c")
    sid = lax.axis_index("s")
    wid = cid * NS + sid
    pltpu.sync_copy(src_hbm.at[wid], src_v)
    pltpu.sync_copy(dst_hbm.at[wid], dst_v)
    pltpu.sync_copy(p_hbm, p_v)
    pltpu.sync_copy(q_hbm, q_v)

    zeros = jnp.zeros((16,), jnp.float32)

    def zbody(i, _):
        ap_v[pl.ds(i * 16, 16)] = zeros
        aq_v[pl.ds(i * 16, 16)] = zeros
        return 0

    lax.fori_loop(0, N_PAD // 16, zbody, 0)

    def body(j, _):
        o = pl.multiple_of(j * 16, 8)
        s_idx = src_v[pl.ds(o, 16)]
        d_idx = dst_v[pl.ds(o, 16)]
        plsc.addupdate_scatter(ap_v, [d_idx], plsc.load_gather(p_v, [s_idx]))
        plsc.addupdate_scatter(aq_v, [d_idx], plsc.load_gather(q_v, [s_idx]))
        return 0

    lax.fori_loop(0, EPW // 16, body, 0)
    pltpu.sync_copy(ap_v, outp_hbm.at[wid])
    pltpu.sync_copy(aq_v, outq_hbm.at[wid])


_BLK = 1024
_GRID = N_PAD // _BLK


_COL = lambda i: (i, 0)
_CSPEC = pl.BlockSpec((_BLK, 1), _COL)
_PSPEC = pl.BlockSpec((NW, _BLK), lambda i: (0, i))
_CSHAPE = jax.ShapeDtypeStruct((N_PAD, 1), jnp.float32)


def _tc_norm_body(w_ref, deg_ref, nrm_ref, p1_ref):
    deg = jnp.sum(deg_ref[...], axis=0)[:, None] + 1.0
    norm = lax.rsqrt(deg)
    nrm_ref[...] = norm
    p1_ref[...] = norm * w_ref[...]


def _tc_norm(weight_pad, deg_parts):
    return pl.pallas_call(
        _tc_norm_body,
        grid=(_GRID,),
        in_specs=[_CSPEC, _PSPEC],
        out_specs=[_CSPEC, _CSPEC],
        out_shape=[_CSHAPE, _CSHAPE],
    )(weight_pad, deg_parts)


def _tc_scal_combine_body(sp_ref, sq_ref, p1_ref, q1_ref, nrm_ref,
                          a1_ref, b1_ref, p2_ref, q2_ref):
    norm = nrm_ref[...]
    a1 = -norm * (jnp.sum(sp_ref[...], axis=0)[:, None] + p1_ref[...])
    b1 = -norm * (jnp.sum(sq_ref[...], axis=0)[:, None] + q1_ref[...])
    a1_ref[...] = a1
    b1_ref[...] = b1
    p2_ref[...] = norm * a1
    q2_ref[...] = norm * b1


def _tc_scal_combine(sp_parts, sq_parts, p1, q1, norm):
    return pl.pallas_call(
        _tc_scal_combine_body,
        grid=(_GRID,),
        in_specs=[_PSPEC, _PSPEC, _CSPEC, _CSPEC, _CSPEC],
        out_specs=[_CSPEC, _CSPEC, _CSPEC, _CSPEC],
        out_shape=[_CSHAPE, _CSHAPE, _CSHAPE, _CSHAPE],
    )(sp_parts, sq_parts, p1, q1, norm)


def _tc_conv1_body(sp_ref, sq_ref, p2_ref, q2_ref, w_ref, nrm_ref, a1_ref,
                   b1_ref, win_ref, bin_ref, cw_ref, cb_ref, cw2_ref,
                   wout_ref, e0_ref, e2_ref, pa_ref, qa_ref):
    norm = nrm_ref[...]
    ah2 = norm * (jnp.sum(sp_ref[...], axis=0)[:, None] + p2_ref[...])
    bh2 = norm * (jnp.sum(sq_ref[...], axis=0)[:, None] + q2_ref[...])
    a2 = -2.0 * ah2 - w_ref[...]
    b2 = -2.0 * bh2 - 1.0
    win, bin_ = win_ref[...], bin_ref[...]
    tx0 = w_ref[...] * win + bin_
    tx1 = a1_ref[...] * win + b1_ref[...] * bin_
    tx2 = a2 * win + b2 * bin_
    rst = (jnp.dot(tx0, cw_ref[0], preferred_element_type=jnp.float32)
           + jnp.dot(tx1, cw_ref[1], preferred_element_type=jnp.float32)
           + jnp.dot(tx2, cw_ref[2], preferred_element_type=jnp.float32)
           + cb_ref[...])
    emb = jnp.where(rst >= 0.0, rst, 0.01 * rst)
    wout = wout_ref[...]
    v0 = jnp.dot(cw2_ref[0], wout, preferred_element_type=jnp.float32)
    v1 = jnp.dot(cw2_ref[1], wout, preferred_element_type=jnp.float32)
    v2 = jnp.dot(cw2_ref[2], wout, preferred_element_type=jnp.float32)
    e0_ref[...] = jnp.dot(emb, v0, preferred_element_type=jnp.float32)
    e1 = jnp.dot(emb, v1, preferred_element_type=jnp.float32)
    e2 = jnp.dot(emb, v2, preferred_element_type=jnp.float32)
    e2_ref[...] = e2
    pa_ref[...] = norm * e1
    qa_ref[...] = norm * e2


def _tc_conv1(sp_parts, sq_parts, p2, q2, weight_pad, norm, a1, b1,
              w_in, b_in, cw, cb, cw2, w_out):
    return pl.pallas_call(
        _tc_conv1_body,
        grid=(_GRID,),
        in_specs=[_PSPEC, _PSPEC, _CSPEC, _CSPEC, _CSPEC, _CSPEC, _CSPEC,
                  _CSPEC,
                  pl.BlockSpec((1, D), lambda i: (0, 0)),
                  pl.BlockSpec((1, D), lambda i: (0, 0)),
                  pl.BlockSpec((K, D, D), lambda i: (0, 0, 0)),
                  pl.BlockSpec((1, D), lambda i: (0, 0)),
                  pl.BlockSpec((K, D, D), lambda i: (0, 0, 0)),
                  pl.BlockSpec((D, 1), lambda i: (0, 0))],
        out_specs=[_CSPEC, _CSPEC, _CSPEC, _CSPEC],
        out_shape=[_CSHAPE, _CSHAPE, _CSHAPE, _CSHAPE],
    )(sp_parts, sq_parts, p2, q2, weight_pad, norm, a1, b1, w_in, b_in,
      cw, cb, cw2, w_out)


def _tc_zcomb_body(sp_ref, sq_ref, pa_ref, qa_ref, nrm_ref,
                   t1_ref, pb_ref):
    norm = nrm_ref[...]
    t1_ref[...] = -norm * (jnp.sum(sp_ref[...], axis=0)[:, None]
                           + pa_ref[...])
    z = -norm * (jnp.sum(sq_ref[...], axis=0)[:, None] + qa_ref[...])
    pb_ref[...] = norm * z


def _tc_zcomb(sp_parts, sq_parts, pa, qa, norm):
    return pl.pallas_call(
        _tc_zcomb_body,
        grid=(_GRID,),
        in_specs=[_PSPEC, _PSPEC, _CSPEC, _CSPEC, _CSPEC],
        out_specs=[_CSPEC, _CSPEC],
        out_shape=[_CSHAPE, _CSHAPE],
    )(sp_parts, sq_parts, pa, qa, norm)


def _tc_logits_body(sp_ref, pb_ref, nrm_ref, e0_ref, e2_ref, t1_ref,
                    cb2_ref, wout_ref, bout_ref, out_ref):
    h2v2 = nrm_ref[...] * (jnp.sum(sp_ref[...], axis=0)[:, None]
                           + pb_ref[...])
    t2v2 = -2.0 * h2v2 - e2_ref[...]
    bias = jnp.dot(cb2_ref[...], wout_ref[...],
                   preferred_element_type=jnp.float32)
    out_ref[...] = (e0_ref[...] + t1_ref[...] + t2v2 + bias
                    + bout_ref[...])


def _tc_logits(sp_parts, pb, norm, e0, e2, t1v1, cb2, w_out, b_out):
    return pl.pallas_call(
        _tc_logits_body,
        grid=(_GRID,),
        in_specs=[_PSPEC, _CSPEC, _CSPEC, _CSPEC, _CSPEC, _CSPEC,
                  pl.BlockSpec((1, D), lambda i: (0, 0)),
                  pl.BlockSpec((D, 1), lambda i: (0, 0)),
                  pl.BlockSpec((1, 1), lambda i: (0, 0))],
        out_specs=_CSPEC,
        out_shape=_CSHAPE,
    )(sp_parts, pb, norm, e0, e2, t1v1, cb2, w_out, b_out)


def kernel(weight, edge_index, W_in, b_in, cheb_w, cheb_b, W_out, b_out):
    src = edge_index[0].reshape(NW, EPW)
    dst_flat = edge_index[1].reshape(NW, EPW)
    dst_deg = edge_index[1].reshape(NW, EPW // 16, 16)
    weight_pad = jnp.pad(weight, (0, N_PAD - N)).reshape(N_PAD, 1)
    w_in = W_in.reshape(1, D)
    b_in = b_in.reshape(1, D)

    deg_parts = _sc_degree(dst_deg)
    norm, p1 = _tc_norm(weight_pad, deg_parts)

    sp1, sq1 = _sc_scalar_sum(src, dst_flat, p1.reshape(N_PAD),
                              norm.reshape(N_PAD))
    a1, b1, p2, q2 = _tc_scal_combine(sp1, sq1, p1, norm, norm)
    sp2, sq2 = _sc_scalar_sum(src, dst_flat, p2.reshape(N_PAD),
                              q2.reshape(N_PAD))
    e0, e2, pa, qa = _tc_conv1(sp2, sq2, p2, q2, weight_pad, norm, a1, b1,
                               w_in, b_in, cheb_w[0], cheb_b[0].reshape(1, D),
                               cheb_w[1], W_out)

    spa, sqa = _sc_scalar_sum(src, dst_flat, pa.reshape(N_PAD),
                              qa.reshape(N_PAD))
    t1v1, pb = _tc_zcomb(spa, sqa, pa, qa, norm)
    spb, _ = _sc_scalar_sum(src, dst_flat, pb.reshape(N_PAD),
                            pb.reshape(N_PAD))
    logits = _tc_logits(spb, pb, norm, e0, e2, t1v1,
                        cheb_b[1].reshape(1, D), W_out, b_out.reshape(1, 1))
    return logits[:N]
